# Initial kernel scaffold; baseline (speedup 1.0000x reference)
import jax
import jax.numpy as jnp
from jax.experimental import pallas as pl


def kernel(x, edge_index, edge_attr, W_enc, b_enc, g1_W, g1_att_src, g1_att_dst, g1_b, g2_W, g2_att_src, g2_att_dst, g2_b, n1_W, n1_b, n2_W, n2_b, e1_W, e1_b, e2_W, e2_b):
    raise NotImplementedError("write your pallas kernel here")


# speedup vs baseline: 13.5111x; 13.5111x over previous
import functools

import jax
import jax.numpy as jnp
from jax import lax
from jax.experimental import pallas as pl
from jax.experimental.pallas import tpu as pltpu
from jax.experimental.pallas import tpu_sc as plsc

N = 100000
E = 1600000
NP = 102400
EP = 1638400
PADN = NP - N
BN = 2048
BE = 8192
K = 128
F32 = jnp.float32
HI = jax.lax.Precision.HIGHEST


def _lrelu(x, s):
    return jnp.maximum(x, s * x)


def _elu(x):
    return jnp.where(x > 0, x, jnp.exp(jnp.minimum(x, 0.0)) - 1.0)


def _packmax(sm, dm):
    pk = jnp.stack([sm, dm])
    return jnp.pad(pk, ((0, 6), (0, 128 - pk.shape[1])))


def _maxacc(ref, pk):
    i = pl.program_id(0)

    @pl.when(i == 0)
    def _():
        ref[...] = pk

    @pl.when(i > 0)
    def _():
        ref[...] = jnp.maximum(ref[...], pk)


def _t1_body(x_ref, wenc_ref, benc_ref, g1w_ref, atts_ref, attd_ref,
             h1_ref, a1_ref, cm_ref):
    h0 = jnp.dot(x_ref[...], wenc_ref[...], precision=HI) + benc_ref[0]
    h1 = jnp.dot(h0, g1w_ref[...], precision=HI)
    h1_ref[...] = h1
    hh = h1.reshape(BN, 4, 16)
    a1s = (hh * atts_ref[:4][None]).sum(-1)
    a1d = (hh * attd_ref[:4][None]).sum(-1)
    a1_ref[...] = jnp.concatenate([a1s, a1d], axis=1)
    _maxacc(cm_ref, _packmax(jnp.max(a1s, 0), jnp.max(a1d, 0)))


def _t1(xp, wencT, benc, g1wT, atts, attd):
    nb = NP // BN
    return pl.pallas_call(
        _t1_body,
        grid=(nb,),
        in_specs=[
            pl.BlockSpec((BN, 128), lambda i: (i, 0)),
            pl.BlockSpec((128, 64), lambda i: (0, 0)),
            pl.BlockSpec((8, 64), lambda i: (0, 0)),
            pl.BlockSpec((64, 64), lambda i: (0, 0)),
            pl.BlockSpec((8, 16), lambda i: (0, 0)),
            pl.BlockSpec((8, 16), lambda i: (0, 0)),
        ],
        out_specs=[
            pl.BlockSpec((BN, 64), lambda i: (i, 0)),
            pl.BlockSpec((BN, 8), lambda i: (i, 0)),
            pl.BlockSpec((8, 128), lambda i: (0, 0)),
        ],
        out_shape=[
            jax.ShapeDtypeStruct((NP, 64), F32),
            jax.ShapeDtypeStruct((NP, 8), F32),
            jax.ShapeDtypeStruct((8, 128), F32),
        ],
    )(xp, wencT, benc, g1wT, atts, attd)


def _t2_body(out1_ref, s1_ref, h1_ref, a1_ref, cm_ref, g1b_ref, g2w_ref,
             atts_ref, attd_ref, h2_ref, a2_ref, cm2_ref):
    cm = cm_ref[...]
    z = cm[0, :4] + cm[1, :4]
    c1 = _lrelu(z, 0.2)
    a1 = a1_ref[...]
    asn, adn = a1[:, :4], a1[:, 4:]
    wself = jnp.exp(_lrelu(asn + adn, 0.2) - c1[None])
    h1 = h1_ref[...]
    num = (out1_ref[...].reshape(BN, 4, 16)
           + wself[:, :, None] * h1.reshape(BN, 4, 16))
    s1 = s1_ref[...]
    den = s1[:, :4] + s1[:, 4:] + wself
    hmid = _elu((num / den[:, :, None]).reshape(BN, 64) + g1b_ref[0])
    h2 = jnp.dot(hmid, g2w_ref[...], precision=HI)
    h2_ref[...] = h2
    a2s = (h2 * atts_ref[0][None]).sum(-1)
    a2d = (h2 * attd_ref[0][None]).sum(-1)
    a2_ref[...] = jnp.concatenate(
        [a2s[:, None], a2d[:, None], jnp.zeros((BN, 6), F32)], axis=1)
    _maxacc(cm2_ref, _packmax(jnp.max(a2s)[None], jnp.max(a2d)[None]))


def _t2(out1n, s1n, h1, a1n, cm1, g1b, g2wT, atts2, attd2):
    nb = NP // BN
    return pl.pallas_call(
        _t2_body,
        grid=(nb,),
        in_specs=[
            pl.BlockSpec((BN, 64), lambda i: (i, 0)),
            pl.BlockSpec((BN, 8), lambda i: (i, 0)),
            pl.BlockSpec((BN, 64), lambda i: (i, 0)),
            pl.BlockSpec((BN, 8), lambda i: (i, 0)),
            pl.BlockSpec((8, 128), lambda i: (0, 0)),
            pl.BlockSpec((8, 64), lambda i: (0, 0)),
            pl.BlockSpec((64, 64), lambda i: (0, 0)),
            pl.BlockSpec((8, 64), lambda i: (0, 0)),
            pl.BlockSpec((8, 64), lambda i: (0, 0)),
        ],
        out_specs=[
            pl.BlockSpec((BN, 64), lambda i: (i, 0)),
            pl.BlockSpec((BN, 8), lambda i: (i, 0)),
            pl.BlockSpec((8, 128), lambda i: (0, 0)),
        ],
        out_shape=[
            jax.ShapeDtypeStruct((NP, 64), F32),
            jax.ShapeDtypeStruct((NP, 8), F32),
            jax.ShapeDtypeStruct((8, 128), F32),
        ],
    )(out1n, s1n, h1, a1n, cm1, g1b, g2wT, atts2, attd2)


def _t3_body(out2_ref, s2_ref, h2_ref, a2_ref, cm_ref, g2b_ref,
             n1w_ref, n1b_ref, n2w_ref, n2b_ref, wsT_ref, wdT_ref,
             npred_ref, ps_ref, pd_ref):
    cm = cm_ref[...]
    c2 = _lrelu(cm[0, 0] + cm[1, 0], 0.2)
    a2 = a2_ref[...]
    asn, adn = a2[:, 0], a2[:, 1]
    wself = jnp.exp(_lrelu(asn + adn, 0.2) - c2)
    num = out2_ref[...] + wself[:, None] * h2_ref[...]
    s2 = s2_ref[...]
    den = s2[:, 0] + s2[:, 4] + wself
    hfin = num / den[:, None] + g2b_ref[0]
    z1 = _lrelu(jnp.dot(hfin, n1w_ref[...], precision=HI) + n1b_ref[0], 0.01)
    npred = jnp.tanh(jnp.dot(z1, n2w_ref[...], precision=HI) + n2b_ref[0])
    npred_ref[...] = npred
    ps_ref[...] = jnp.dot(hfin, wsT_ref[...], precision=HI)
    pd_ref[...] = jnp.dot(hfin, wdT_ref[...], precision=HI)


def _t3(out2n, s2n, h2, a2n, cm2, g2b, n1wT, n1b, n2wTp, n2bp, wsT, wdT):
    nb = NP // BN
    return pl.pallas_call(
        _t3_body,
        grid=(nb,),
        in_specs=[
            pl.BlockSpec((BN, 64), lambda i: (i, 0)),
            pl.BlockSpec((BN, 8), lambda i: (i, 0)),
            pl.BlockSpec((BN, 64), lambda i: (i, 0)),
            pl.BlockSpec((BN, 8), lambda i: (i, 0)),
            pl.BlockSpec((8, 128), lambda i: (0, 0)),
            pl.BlockSpec((8, 64), lambda i: (0, 0)),
            pl.BlockSpec((64, 64), lambda i: (0, 0)),
            pl.BlockSpec((8, 64), lambda i: (0, 0)),
            pl.BlockSpec((64, 128), lambda i: (0, 0)),
            pl.BlockSpec((8, 128), lambda i: (0, 0)),
            pl.BlockSpec((64, 64), lambda i: (0, 0)),
            pl.BlockSpec((64, 64), lambda i: (0, 0)),
        ],
        out_specs=[
            pl.BlockSpec((BN, 128), lambda i: (i, 0)),
            pl.BlockSpec((BN, 64), lambda i: (i, 0)),
            pl.BlockSpec((BN, 64), lambda i: (i, 0)),
        ],
        out_shape=[
            jax.ShapeDtypeStruct((NP, 128), F32),
            jax.ShapeDtypeStruct((NP, 64), F32),
            jax.ShapeDtypeStruct((NP, 64), F32),
        ],
    )(out2n, s2n, h2, a2n, cm2, g2b, n1wT, n1b, n2wTp, n2bp, wsT, wdT)


def _t4_body(e1pre_ref, attr_ref, weT_ref, e1b_ref, e2w_ref, e2b_ref,
             out_ref):
    e1 = (e1pre_ref[...]
          + jnp.dot(attr_ref[...], weT_ref[:4], precision=HI) + e1b_ref[0])
    a = _lrelu(e1, 0.01)
    z = jnp.dot(a, e2w_ref[...], precision=HI)[:, 0] + e2b_ref[0, 0]
    out_ref[...] = jnp.tanh(z).reshape(BE // 128, 128)


def _t4(e1pre, attrp, weT, e1b, e2wTp, e2bp):
    nb = EP // BE
    return pl.pallas_call(
        _t4_body,
        grid=(nb,),
        in_specs=[
            pl.BlockSpec((BE, 64), lambda i: (i, 0)),
            pl.BlockSpec((BE, 4), lambda i: (i, 0)),
            pl.BlockSpec((8, 64), lambda i: (0, 0)),
            pl.BlockSpec((8, 64), lambda i: (0, 0)),
            pl.BlockSpec((64, 128), lambda i: (0, 0)),
            pl.BlockSpec((8, 128), lambda i: (0, 0)),
        ],
        out_specs=pl.BlockSpec((BE // 128, 128), lambda i: (i, 0)),
        out_shape=jax.ShapeDtypeStruct((EP // 128, 128), F32),
    )(e1pre, attrp, weT, e1b, e2wTp, e2bp)


def _mesh():
    return plsc.VectorSubcoreMesh(core_axis_name="c", subcore_axis_name="s")


def _zero_fill(zbuf, nvec):
    def zb(i, _):
        zbuf[pl.ds(i * 16, 16)] = jnp.zeros((16,), F32)
        return 0
    lax.fori_loop(0, nvec, zb, 0)


def _build_idx(dst_ref, src_ref, off, nvec):
    def bi(i, _):
        sl = pl.ds(i * 16, 16)
        dst_ref[sl] = src_ref[sl] + off
        return 0
    lax.fori_loop(0, nvec, bi, 0)


def _wcompute(w_ref, as_ref, ad_ref, ch, nvec):
    def wc(i, _):
        sl = pl.ds(i * 16, 16)
        z = as_ref[sl] + ad_ref[sl]
        w_ref[sl] = jnp.exp(jnp.maximum(z, 0.2 * z) - ch)
        return 0
    lax.fori_loop(0, nvec, wc, 0)


def _make_s1(heads):
    soffs = [h * NP for h in range(heads)] if heads == 4 else [0]
    doffs = [(4 + h) * NP for h in range(heads)] if heads == 4 else [NP]

    @functools.partial(
        pl.kernel,
        mesh=_mesh(),
        compiler_params=pltpu.CompilerParams(use_tc_tiling_on_sc=False),
        out_type=jax.ShapeDtypeStruct((8 * NP,), F32),
        scratch_types=[
            pltpu.VMEM((8, 128), F32),
            pltpu.VMEM((K,), jnp.int32),
            pltpu.VMEM((K,), jnp.int32),
            pltpu.VMEM((K,), jnp.int32),
            pltpu.VMEM((K,), jnp.int32),
            pltpu.VMEM((K,), F32),
            pltpu.VMEM((K,), F32),
            pltpu.VMEM((K,), F32),
            pltpu.VMEM((1600,), F32),
            pltpu.VMEM_SHARED((4 * NP,), F32),
            pltpu.SemaphoreType.DMA,
        ],
    )
    def s1(src_hbm, dst_hbm, a_hbm, cm_hbm, sout_hbm,
           cm_v, idx_s, idx_d, gi1, gi2, as_v, ad_v, w_v, zbuf, sacc, sem):
        c = lax.axis_index("c")
        t = lax.axis_index("s")
        pltpu.sync_copy(cm_hbm, cm_v)
        r0 = cm_v[0, pl.ds(0, 16)]
        r1 = cm_v[1, pl.ds(0, 16)]
        chs = [_lrelu(r0[h] + r1[h], 0.2) for h in range(heads)]

        _zero_fill(zbuf, 100)
        zlen = (4 * NP) // 16 // 1600

        def zc(i, _):
            pltpu.sync_copy(
                zbuf, sacc.at[pl.ds(t * 25600 + i * 1600, 1600)])
            return 0
        lax.fori_loop(0, zlen, zc, 0)
        plsc.subcore_barrier()

        ecore = EP // 2
        etile = ecore // 16
        base0 = c * ecore + t * etile

        def step(i, _):
            base = base0 + i * K
            pltpu.sync_copy(src_hbm.at[pl.ds(base, K)], idx_s)
            pltpu.sync_copy(dst_hbm.at[pl.ds(base, K)], idx_d)
            for h in range(heads):
                _build_idx(gi1, idx_s, soffs[h], K // 16)
                _build_idx(gi2, idx_d, doffs[h], K // 16)
                pltpu.async_copy(a_hbm.at[gi1], as_v, sem).wait()
                pltpu.async_copy(a_hbm.at[gi2], ad_v, sem).wait()
                _wcompute(w_v, as_v, ad_v, chs[h], K // 16)
                _build_idx(gi1, idx_d, soffs[h], K // 16)
                pltpu.sync_copy(w_v, sacc.at[gi1], add=True)
            return 0
        lax.fori_loop(0, etile // K, step, 0)
        plsc.subcore_barrier()

        flen = (4 * NP) // 16 if heads == 4 else NP // 16
        pltpu.sync_copy(
            sacc.at[pl.ds(t * flen, flen)],
            sout_hbm.at[pl.ds(c * 4 * NP + t * flen, flen)])

    return s1


def _make_s2(heads):
    @functools.partial(
        pl.kernel,
        mesh=_mesh(),
        compiler_params=pltpu.CompilerParams(use_tc_tiling_on_sc=False),
        out_type=jax.ShapeDtypeStruct((4 * NP, 16), F32),
        scratch_types=[
            pltpu.VMEM((8, 128), F32),
            pltpu.VMEM((K,), jnp.int32),
            pltpu.VMEM((K,), jnp.int32),
            pltpu.VMEM((K,), jnp.int32),
            pltpu.VMEM((K,), jnp.int32),
            pltpu.VMEM((K,), F32),
            pltpu.VMEM((K,), F32),
            pltpu.VMEM((K,), F32),
            pltpu.VMEM((K, 16), F32),
            pltpu.VMEM((400, 16), F32),
            pltpu.VMEM_SHARED((NP, 16), F32),
            pltpu.SemaphoreType.DMA,
        ],
    )
    def s2(src_hbm, dst_hbm, a_hbm, tab_hbm, cm_hbm, out_hbm,
           cm_v, idx_s, idx_d, gi1, gi2, as_v, ad_v, w_v, rows, zbuf,
           acc, sem):
        c = lax.axis_index("c")
        t = lax.axis_index("s")
        pltpu.sync_copy(cm_hbm, cm_v)
        r0 = cm_v[0, pl.ds(0, 16)]
        r1 = cm_v[1, pl.ds(0, 16)]

        def zrow(i, _):
            zbuf[i, :] = jnp.zeros((16,), F32)
            return 0
        lax.fori_loop(0, 400, zrow, 0)

        etile = EP // 16

        chs = [_lrelu(r0[h] + r1[h], 0.2) for h in range(4)]

        for b in range(2):
            j = 2 * c + b
            if heads == 4:
                ch = jnp.where(c == 0, chs[b], chs[2 + b])
                aoff_s, aoff_d = j * NP, (4 + j) * NP
            else:
                ch = chs[0]
                aoff_s, aoff_d = 0, NP

            def zc(i, _):
                pltpu.sync_copy(zbuf, acc.at[pl.ds(t * 6400 + i * 400, 400)])
                return 0
            lax.fori_loop(0, 16, zc, 0)
            plsc.subcore_barrier()

            def step(i, _):
                base = t * etile + i * K
                pltpu.sync_copy(src_hbm.at[pl.ds(base, K)], idx_s)
                pltpu.sync_copy(dst_hbm.at[pl.ds(base, K)], idx_d)
                _build_idx(gi1, idx_s, aoff_s, K // 16)
                _build_idx(gi2, idx_d, aoff_d, K // 16)
                pltpu.async_copy(a_hbm.at[gi1], as_v, sem).wait()
                pltpu.async_copy(a_hbm.at[gi2], ad_v, sem).wait()
                _wcompute(w_v, as_v, ad_v, ch, K // 16)
                _build_idx(gi1, idx_s, j * NP, K // 16)
                pltpu.async_copy(tab_hbm.at[gi1], rows, sem).wait()

                def sc(g, _):
                    w16 = w_v[pl.ds(g * 16, 16)]
                    for e0 in range(16):
                        e = g * 16 + e0
                        rows[e, :] = rows[e, :] * w16[e0]
                    return 0
                lax.fori_loop(0, K // 16, sc, 0)
                pltpu.sync_copy(rows, acc.at[idx_d], add=True)
                return 0
            lax.fori_loop(0, etile // K, step, 0)
            plsc.subcore_barrier()

            pltpu.sync_copy(
                acc.at[pl.ds(t * 6400, 6400)],
                out_hbm.at[pl.ds(j * NP + t * 6400, 6400)])
            plsc.subcore_barrier()

    return s2


def _make_s3():
    @functools.partial(
        pl.kernel,
        mesh=_mesh(),
        compiler_params=pltpu.CompilerParams(use_tc_tiling_on_sc=False),
        out_type=jax.ShapeDtypeStruct((EP, 64), F32),
        scratch_types=[
            pltpu.VMEM((K,), jnp.int32),
            pltpu.VMEM((K,), jnp.int32),
            pltpu.VMEM((K, 64), F32),
            pltpu.VMEM((K, 64), F32),
            pltpu.SemaphoreType.DMA,
        ],
    )
    def s3(src_hbm, dst_hbm, ps_hbm, pd_hbm, out_hbm,
           idx_s, idx_d, ps_v, pd_v, sem):
        c = lax.axis_index("c")
        t = lax.axis_index("s")
        etile = EP // 32
        base0 = c * (EP // 2) + t * etile

        def step(i, _):
            base = base0 + i * K
            pltpu.sync_copy(src_hbm.at[pl.ds(base, K)], idx_s)
            pltpu.sync_copy(dst_hbm.at[pl.ds(base, K)], idx_d)
            pltpu.async_copy(ps_hbm.at[idx_s], ps_v, sem).wait()
            pltpu.async_copy(pd_hbm.at[idx_d], pd_v, sem).wait()

            def ad(e, _):
                for k in range(4):
                    sl = pl.ds(k * 16, 16)
                    ps_v[e, sl] = ps_v[e, sl] + pd_v[e, sl]
                return 0
            lax.fori_loop(0, K, ad, 0)
            pltpu.sync_copy(ps_v, out_hbm.at[pl.ds(base, K)])
            return 0
        lax.fori_loop(0, etile // K, step, 0)

    return s3


def _to_headmajor(h):
    return h.reshape(NP, 4, 16).transpose(1, 0, 2).reshape(4 * NP, 16)


def _from_headmajor(o):
    return o.reshape(4, NP, 16).transpose(1, 0, 2).reshape(NP, 64)


def kernel(x, edge_index, edge_attr, W_enc, b_enc, g1_W, g1_att_src,
           g1_att_dst, g1_b, g2_W, g2_att_src, g2_att_dst, g2_b, n1_W,
           n1_b, n2_W, n2_b, e1_W, e1_b, e2_W, e2_b):
    src0 = edge_index[0]
    dst0 = edge_index[1]
    r = jnp.arange(EP - E, dtype=jnp.int32)
    srcp = jnp.concatenate([src0, N + r % PADN])
    dstp = jnp.concatenate([dst0, N + (r + PADN // 2) % PADN])
    xp = jnp.pad(x, ((0, NP - N), (0, 0)))
    attrp = jnp.pad(edge_attr, ((0, EP - E), (0, 0)))

    def b8(v):
        return jnp.broadcast_to(v[None], (8, v.shape[0]))

    def p8(m):
        return jnp.pad(m, ((0, 8 - m.shape[0]), (0, 0)))

    h1, a1n, cm1 = _t1(xp, W_enc.T, b8(b_enc), g1_W.T,
                       p8(g1_att_src), p8(g1_att_dst))
    a1f = a1n.T.reshape(-1)
    s1 = _make_s1(4)(srcp, dstp, a1f, cm1)
    out1 = _make_s2(4)(srcp, dstp, a1f, _to_headmajor(h1), cm1)

    h2, a2n, cm2 = _t2(_from_headmajor(out1), s1.reshape(8, NP).T, h1, a1n,
                       cm1, b8(g1_b), g2_W.T, b8(g2_att_src[0]),
                       b8(g2_att_dst[0]))
    a2f = a2n.T.reshape(-1)
    s2v = _make_s1(1)(srcp, dstp, a2f, cm2)
    out2 = _make_s2(1)(srcp, dstp, a2f, _to_headmajor(h2), cm2)

    n2wTp = jnp.pad(n2_W.T, ((0, 0), (0, 126)))
    n2bp = jnp.pad(n2_b[None], ((0, 7), (0, 126)))
    wsT = e1_W[:, :64].T
    wdT = e1_W[:, 64:128].T
    npred_full, ps, pd = _t3(
        _from_headmajor(out2), s2v.reshape(8, NP).T, h2, a2n, cm2,
        b8(g2_b), n1_W.T, b8(n1_b), n2wTp, n2bp, wsT, wdT)

    e1pre = _make_s3()(srcp, dstp, ps, pd)
    weT = p8(e1_W[:, 128:].T)
    e2wTp = jnp.pad(e2_W.T, ((0, 0), (0, 127)))
    e2bp = jnp.pad(e2_b[None], ((0, 7), (0, 127)))
    epred_full = _t4(e1pre, attrp, weT, b8(e1_b), e2wTp, e2bp)

    edge_pred = epred_full.reshape(-1)[:E]
    node_pred = npred_full[:N, :2]
    return (edge_pred, node_pred)


# speedup vs baseline: 30.3673x; 2.2476x over previous
import functools

import jax
import jax.numpy as jnp
from jax import lax
from jax.experimental import pallas as pl
from jax.experimental.pallas import tpu as pltpu
from jax.experimental.pallas import tpu_sc as plsc

N = 100000
E = 1600000
NP = 102400
EP = 1638400
PADN = NP - N
BN = 2048
BE = 8192
K = 128
F32 = jnp.float32
HI = jax.lax.Precision.HIGHEST


def _lrelu(x, s):
    return jnp.maximum(x, s * x)


def _elu(x):
    return jnp.where(x > 0, x, jnp.exp(jnp.minimum(x, 0.0)) - 1.0)


def _packmax(sm, dm):
    pk = jnp.stack([sm, dm])
    return jnp.pad(pk, ((0, 6), (0, 128 - pk.shape[1])))


def _maxacc(ref, pk):
    i = pl.program_id(0)

    @pl.when(i == 0)
    def _():
        ref[...] = pk

    @pl.when(i > 0)
    def _():
        ref[...] = jnp.maximum(ref[...], pk)


def _t1_body(x_ref, wenc_ref, benc_ref, g1w_ref, atts_ref, attd_ref,
             h1_ref, a1_ref, cm_ref):
    h0 = jnp.dot(x_ref[...], wenc_ref[...], precision=HI) + benc_ref[0]
    h1 = jnp.dot(h0, g1w_ref[...], precision=HI)
    h1_ref[...] = h1
    hh = h1.reshape(BN, 4, 16)
    a1s = (hh * atts_ref[:4][None]).sum(-1)
    a1d = (hh * attd_ref[:4][None]).sum(-1)
    a1_ref[...] = jnp.concatenate([a1s, a1d], axis=1)
    _maxacc(cm_ref, _packmax(jnp.max(a1s, 0), jnp.max(a1d, 0)))


def _t1(xp, wencT, benc, g1wT, atts, attd):
    nb = NP // BN
    return pl.pallas_call(
        _t1_body,
        grid=(nb,),
        in_specs=[
            pl.BlockSpec((BN, 128), lambda i: (i, 0)),
            pl.BlockSpec((128, 64), lambda i: (0, 0)),
            pl.BlockSpec((8, 64), lambda i: (0, 0)),
            pl.BlockSpec((64, 64), lambda i: (0, 0)),
            pl.BlockSpec((8, 16), lambda i: (0, 0)),
            pl.BlockSpec((8, 16), lambda i: (0, 0)),
        ],
        out_specs=[
            pl.BlockSpec((BN, 64), lambda i: (i, 0)),
            pl.BlockSpec((BN, 8), lambda i: (i, 0)),
            pl.BlockSpec((8, 128), lambda i: (0, 0)),
        ],
        out_shape=[
            jax.ShapeDtypeStruct((NP, 64), F32),
            jax.ShapeDtypeStruct((NP, 8), F32),
            jax.ShapeDtypeStruct((8, 128), F32),
        ],
    )(xp, wencT, benc, g1wT, atts, attd)


def _t2_body(out1_ref, s1_ref, h1_ref, a1_ref, cm_ref, g1b_ref, g2w_ref,
             atts_ref, attd_ref, h2_ref, a2_ref, cm2_ref):
    cm = cm_ref[...]
    z = cm[0, :4] + cm[1, :4]
    c1 = _lrelu(z, 0.2)
    a1 = a1_ref[...]
    asn, adn = a1[:, :4], a1[:, 4:]
    wself = jnp.exp(_lrelu(asn + adn, 0.2) - c1[None])
    h1 = h1_ref[...]
    num = (out1_ref[...].reshape(BN, 4, 16)
           + wself[:, :, None] * h1.reshape(BN, 4, 16))
    s1 = s1_ref[...]
    den = s1[:, :4] + wself
    hmid = _elu((num / den[:, :, None]).reshape(BN, 64) + g1b_ref[0])
    h2 = jnp.dot(hmid, g2w_ref[...], precision=HI)
    h2_ref[...] = h2
    a2s = (h2 * atts_ref[0][None]).sum(-1)
    a2d = (h2 * attd_ref[0][None]).sum(-1)
    a2_ref[...] = jnp.concatenate(
        [a2s[:, None], a2d[:, None], jnp.zeros((BN, 6), F32)], axis=1)
    _maxacc(cm2_ref, _packmax(jnp.max(a2s)[None], jnp.max(a2d)[None]))


def _t2(out1n, s1n, h1, a1n, cm1, g1b, g2wT, atts2, attd2):
    nb = NP // BN
    return pl.pallas_call(
        _t2_body,
        grid=(nb,),
        in_specs=[
            pl.BlockSpec((BN, 64), lambda i: (i, 0)),
            pl.BlockSpec((BN, 8), lambda i: (i, 0)),
            pl.BlockSpec((BN, 64), lambda i: (i, 0)),
            pl.BlockSpec((BN, 8), lambda i: (i, 0)),
            pl.BlockSpec((8, 128), lambda i: (0, 0)),
            pl.BlockSpec((8, 64), lambda i: (0, 0)),
            pl.BlockSpec((64, 64), lambda i: (0, 0)),
            pl.BlockSpec((8, 64), lambda i: (0, 0)),
            pl.BlockSpec((8, 64), lambda i: (0, 0)),
        ],
        out_specs=[
            pl.BlockSpec((BN, 64), lambda i: (i, 0)),
            pl.BlockSpec((BN, 8), lambda i: (i, 0)),
            pl.BlockSpec((8, 128), lambda i: (0, 0)),
        ],
        out_shape=[
            jax.ShapeDtypeStruct((NP, 64), F32),
            jax.ShapeDtypeStruct((NP, 8), F32),
            jax.ShapeDtypeStruct((8, 128), F32),
        ],
    )(out1n, s1n, h1, a1n, cm1, g1b, g2wT, atts2, attd2)


def _t3_body(out2_ref, s2_ref, h2_ref, a2_ref, cm_ref, g2b_ref,
             n1w_ref, n1b_ref, n2w_ref, n2b_ref, wsT_ref, wdT_ref,
             npred_ref, ps_ref, pd_ref):
    cm = cm_ref[...]
    c2 = _lrelu(cm[0, 0] + cm[1, 0], 0.2)
    a2 = a2_ref[...]
    asn, adn = a2[:, 0], a2[:, 1]
    wself = jnp.exp(_lrelu(asn + adn, 0.2) - c2)
    num = out2_ref[...] + wself[:, None] * h2_ref[...]
    s2 = s2_ref[...]
    den = s2[:, 0] + wself
    hfin = num / den[:, None] + g2b_ref[0]
    z1 = _lrelu(jnp.dot(hfin, n1w_ref[...], precision=HI) + n1b_ref[0], 0.01)
    npred = jnp.tanh(jnp.dot(z1, n2w_ref[...], precision=HI) + n2b_ref[0])
    npred_ref[...] = npred
    ps_ref[...] = jnp.dot(hfin, wsT_ref[...], precision=HI)
    pd_ref[...] = jnp.dot(hfin, wdT_ref[...], precision=HI)


def _t3(out2n, s2n, h2, a2n, cm2, g2b, n1wT, n1b, n2wTp, n2bp, wsT, wdT):
    nb = NP // BN
    return pl.pallas_call(
        _t3_body,
        grid=(nb,),
        in_specs=[
            pl.BlockSpec((BN, 64), lambda i: (i, 0)),
            pl.BlockSpec((BN, 8), lambda i: (i, 0)),
            pl.BlockSpec((BN, 64), lambda i: (i, 0)),
            pl.BlockSpec((BN, 8), lambda i: (i, 0)),
            pl.BlockSpec((8, 128), lambda i: (0, 0)),
            pl.BlockSpec((8, 64), lambda i: (0, 0)),
            pl.BlockSpec((64, 64), lambda i: (0, 0)),
            pl.BlockSpec((8, 64), lambda i: (0, 0)),
            pl.BlockSpec((64, 128), lambda i: (0, 0)),
            pl.BlockSpec((8, 128), lambda i: (0, 0)),
            pl.BlockSpec((64, 64), lambda i: (0, 0)),
            pl.BlockSpec((64, 64), lambda i: (0, 0)),
        ],
        out_specs=[
            pl.BlockSpec((BN, 128), lambda i: (i, 0)),
            pl.BlockSpec((BN, 64), lambda i: (i, 0)),
            pl.BlockSpec((BN, 64), lambda i: (i, 0)),
        ],
        out_shape=[
            jax.ShapeDtypeStruct((NP, 128), F32),
            jax.ShapeDtypeStruct((NP, 64), F32),
            jax.ShapeDtypeStruct((NP, 64), F32),
        ],
    )(out2n, s2n, h2, a2n, cm2, g2b, n1wT, n1b, n2wTp, n2bp, wsT, wdT)


def _t4_body(e1pre_ref, attr_ref, weT_ref, e1b_ref, e2w_ref, e2b_ref,
             out_ref):
    e1 = (e1pre_ref[...]
          + jnp.dot(attr_ref[...], weT_ref[:4], precision=HI) + e1b_ref[0])
    a = _lrelu(e1, 0.01)
    z = jnp.dot(a, e2w_ref[...], precision=HI)[:, 0] + e2b_ref[0, 0]
    out_ref[...] = jnp.tanh(z).reshape(BE // 128, 128)


def _t4(e1pre, attrp, weT, e1b, e2wTp, e2bp):
    nb = EP // BE
    return pl.pallas_call(
        _t4_body,
        grid=(nb,),
        in_specs=[
            pl.BlockSpec((BE, 64), lambda i: (i, 0)),
            pl.BlockSpec((BE, 4), lambda i: (i, 0)),
            pl.BlockSpec((8, 64), lambda i: (0, 0)),
            pl.BlockSpec((8, 64), lambda i: (0, 0)),
            pl.BlockSpec((64, 128), lambda i: (0, 0)),
            pl.BlockSpec((8, 128), lambda i: (0, 0)),
        ],
        out_specs=pl.BlockSpec((BE // 128, 128), lambda i: (i, 0)),
        out_shape=jax.ShapeDtypeStruct((EP // 128, 128), F32),
    )(e1pre, attrp, weT, e1b, e2wTp, e2bp)


def _mesh():
    return plsc.VectorSubcoreMesh(core_axis_name="c", subcore_axis_name="s")


def _zero_fill(zbuf, nvec):
    def zb(i, _):
        zbuf[pl.ds(i * 16, 16)] = jnp.zeros((16,), F32)
        return 0
    lax.fori_loop(0, nvec, zb, 0)


def _build_idx(dst_ref, src_ref, off, nvec):
    def bi(i, _):
        sl = pl.ds(i * 16, 16)
        dst_ref[sl] = src_ref[sl] + off
        return 0
    lax.fori_loop(0, nvec, bi, 0)


def _wcompute(w_ref, as_ref, ad_ref, ch, nvec):
    def wc(i, _):
        sl = pl.ds(i * 16, 16)
        z = as_ref[sl] + ad_ref[sl]
        w_ref[sl] = jnp.exp(jnp.maximum(z, 0.2 * z) - ch)
        return 0
    lax.fori_loop(0, nvec, wc, 0)


P = 4


def _build_idx2(dst_ref, src_ref, srcbase, off, nvec):
    def bi(g, _):
        dst_ref[pl.ds(g * 16, 16)] = src_ref[pl.ds(srcbase + g * 16, 16)] + off
        return 0
    lax.fori_loop(0, nvec, bi, 0)


def _make_s2(heads):
    scr = [pltpu.VMEM((8, 128), F32),
           pltpu.VMEM((P * K,), jnp.int32),
           pltpu.VMEM((P * K,), jnp.int32)]
    scr += [pltpu.VMEM((K,), jnp.int32) for _ in range(2 * P)]
    scr += [pltpu.VMEM((K,), jnp.int32) for _ in range(P)]
    scr += [pltpu.VMEM((K,), F32) for _ in range(3 * P)]
    scr += [pltpu.VMEM((K, 16), F32) for _ in range(P)]
    scr += [pltpu.VMEM((400, 16), F32), pltpu.VMEM((400,), F32)]
    scr += [pltpu.VMEM_SHARED((NP, 16), F32), pltpu.VMEM_SHARED((NP,), F32)]
    scr += [pltpu.SemaphoreType.DMA, pltpu.SemaphoreType.DMA,
            pltpu.SemaphoreType.DMA]

    @functools.partial(
        pl.kernel,
        mesh=_mesh(),
        compiler_params=pltpu.CompilerParams(use_tc_tiling_on_sc=False),
        out_type=[jax.ShapeDtypeStruct((4 * NP, 16), F32),
                  jax.ShapeDtypeStruct((8 * NP,), F32)],
        scratch_types=scr,
    )
    def s2(src_hbm, dst_hbm, a_hbm, tab_hbm, cm_hbm, out_hbm, sout_hbm,
           *s):
        cm_v, ixs, ixd = s[0], s[1], s[2]
        gia = s[3:3 + P]
        gid = s[3 + P:3 + 2 * P]
        gir = s[3 + 2 * P:3 + 3 * P]
        as_v = s[3 + 3 * P:3 + 4 * P]
        ad_v = s[3 + 4 * P:3 + 5 * P]
        w_v = s[3 + 5 * P:3 + 6 * P]
        rows = s[3 + 6 * P:3 + 7 * P]
        zb2, zb1, acc, sacc, semI, semG, semS = s[3 + 7 * P:]

        c = lax.axis_index("c")
        t = lax.axis_index("s")
        pltpu.sync_copy(cm_hbm, cm_v)
        r0 = cm_v[0, pl.ds(0, 16)]
        r1 = cm_v[1, pl.ds(0, 16)]
        chs = [_lrelu(r0[h] + r1[h], 0.2) for h in range(4)]

        def zrow(i, _):
            zb2[i, :] = jnp.zeros((16,), F32)
            zb1[pl.ds(0, 16)] = jnp.zeros((16,), F32)
            return 0
        lax.fori_loop(0, 400, zrow, 0)
        _zero_fill(zb1, 25)

        etile = EP // 16
        nit = etile // (P * K)

        for b in range(2):
            j = 2 * c + b
            if heads == 4:
                ch = jnp.where(c == 0, chs[b], chs[2 + b])
                aoff_s, aoff_d = j * NP, (4 + j) * NP
                do_s = True
                sflush = j * NP
            else:
                ch = chs[0]
                aoff_s, aoff_d = 0, NP
                do_s = b == 0
                sflush = c * NP

            def zc(i, _):
                pltpu.sync_copy(zb2, acc.at[pl.ds(t * 6400 + i * 400, 400)])
                if do_s:
                    pltpu.sync_copy(
                        zb1, sacc.at[pl.ds(t * 6400 + i * 400, 400)])
                return 0
            lax.fori_loop(0, 16, zc, 0)
            plsc.subcore_barrier()

            def step(i, _):
                base = t * etile + i * (P * K)
                h1 = pltpu.async_copy(src_hbm.at[pl.ds(base, P * K)], ixs,
                                      semI)
                h2 = pltpu.async_copy(dst_hbm.at[pl.ds(base, P * K)], ixd,
                                      semI)
                h1.wait()
                h2.wait()
                for p in range(P):
                    _build_idx2(gia[p], ixs, p * K, aoff_s, K // 16)
                    _build_idx2(gid[p], ixd, p * K, aoff_d, K // 16)
                    _build_idx2(gir[p], ixs, p * K, j * NP, K // 16)
                ha, hd, hr = [], [], []
                for p in range(P):
                    ha.append(pltpu.async_copy(a_hbm.at[gia[p]], as_v[p],
                                               semG))
                    hd.append(pltpu.async_copy(a_hbm.at[gid[p]], ad_v[p],
                                               semG))
                    hr.append(pltpu.async_copy(tab_hbm.at[gir[p]], rows[p],
                                               semG))
                hs = []
                for p in range(P):
                    ha[p].wait()
                    hd[p].wait()
                    _wcompute(w_v[p], as_v[p], ad_v[p], ch, K // 16)
                    _build_idx2(gia[p], ixd, p * K, 0, K // 16)
                    hr[p].wait()

                    def sc(g, _):
                        w16 = w_v[p][pl.ds(g * 16, 16)]
                        for e0 in range(16):
                            e = g * 16 + e0
                            rows[p][e, :] = rows[p][e, :] * w16[e0]
                        return 0
                    lax.fori_loop(0, K // 16, sc, 0)
                    hs.append(pltpu.async_copy(rows[p], acc.at[gia[p]],
                                               semS, add=True))
                    if do_s:
                        hs.append(pltpu.async_copy(w_v[p], sacc.at[gia[p]],
                                                   semS, add=True))
                for h in hs:
                    h.wait()
                return 0
            lax.fori_loop(0, nit, step, 0)
            plsc.subcore_barrier()

            pltpu.sync_copy(
                acc.at[pl.ds(t * 6400, 6400)],
                out_hbm.at[pl.ds(j * NP + t * 6400, 6400)])
            if do_s:
                pltpu.sync_copy(
                    sacc.at[pl.ds(t * 6400, 6400)],
                    sout_hbm.at[pl.ds(sflush + t * 6400, 6400)])
            plsc.subcore_barrier()

    return s2


def _make_s3():
    scr = [pltpu.VMEM((P * K,), jnp.int32), pltpu.VMEM((P * K,), jnp.int32)]
    scr += [pltpu.VMEM((K,), jnp.int32) for _ in range(2 * P)]
    scr += [pltpu.VMEM((K, 64), F32) for _ in range(2 * P)]
    scr += [pltpu.SemaphoreType.DMA, pltpu.SemaphoreType.DMA,
            pltpu.SemaphoreType.DMA]

    @functools.partial(
        pl.kernel,
        mesh=_mesh(),
        compiler_params=pltpu.CompilerParams(use_tc_tiling_on_sc=False),
        out_type=jax.ShapeDtypeStruct((EP, 64), F32),
        scratch_types=scr,
    )
    def s3(src_hbm, dst_hbm, ps_hbm, pd_hbm, out_hbm, *s):
        ixs, ixd = s[0], s[1]
        gia = s[2:2 + P]
        gid = s[2 + P:2 + 2 * P]
        ps_v = s[2 + 2 * P:2 + 3 * P]
        pd_v = s[2 + 3 * P:2 + 4 * P]
        semI, semG, semS = s[2 + 4 * P:]
        c = lax.axis_index("c")
        t = lax.axis_index("s")
        etile = EP // 32
        base0 = c * (EP // 2) + t * etile
        nit = etile // (P * K)

        def step(i, _):
            base = base0 + i * (P * K)
            h1 = pltpu.async_copy(src_hbm.at[pl.ds(base, P * K)], ixs, semI)
            h2 = pltpu.async_copy(dst_hbm.at[pl.ds(base, P * K)], ixd, semI)
            h1.wait()
            h2.wait()
            for p in range(P):
                _build_idx2(gia[p], ixs, p * K, 0, K // 16)
                _build_idx2(gid[p], ixd, p * K, 0, K // 16)
            ha, hd = [], []
            for p in range(P):
                ha.append(pltpu.async_copy(ps_hbm.at[gia[p]], ps_v[p], semG))
                hd.append(pltpu.async_copy(pd_hbm.at[gid[p]], pd_v[p], semG))
            hs = []
            for p in range(P):
                ha[p].wait()
                hd[p].wait()

                def ad(e, _):
                    for k in range(4):
                        sl = pl.ds(k * 16, 16)
                        ps_v[p][e, sl] = ps_v[p][e, sl] + pd_v[p][e, sl]
                    return 0
                lax.fori_loop(0, K, ad, 0)
                hs.append(pltpu.async_copy(
                    ps_v[p], out_hbm.at[pl.ds(base + p * K, K)], semS))
            for h in hs:
                h.wait()
            return 0
        lax.fori_loop(0, nit, step, 0)

    return s3


def _to_headmajor(h):
    return h.reshape(NP, 4, 16).transpose(1, 0, 2).reshape(4 * NP, 16)


def _from_headmajor(o):
    return o.reshape(4, NP, 16).transpose(1, 0, 2).reshape(NP, 64)


def kernel(x, edge_index, edge_attr, W_enc, b_enc, g1_W, g1_att_src,
           g1_att_dst, g1_b, g2_W, g2_att_src, g2_att_dst, g2_b, n1_W,
           n1_b, n2_W, n2_b, e1_W, e1_b, e2_W, e2_b):
    src0 = edge_index[0]
    dst0 = edge_index[1]
    r = jnp.arange(EP - E, dtype=jnp.int32)
    srcp = jnp.concatenate([src0, N + r % PADN])
    dstp = jnp.concatenate([dst0, N + (r + PADN // 2) % PADN])
    xp = jnp.pad(x, ((0, NP - N), (0, 0)))
    attrp = jnp.pad(edge_attr, ((0, EP - E), (0, 0)))

    def b8(v):
        return jnp.broadcast_to(v[None], (8, v.shape[0]))

    def p8(m):
        return jnp.pad(m, ((0, 8 - m.shape[0]), (0, 0)))

    h1, a1n, cm1 = _t1(xp, W_enc.T, b8(b_enc), g1_W.T,
                       p8(g1_att_src), p8(g1_att_dst))
    a1f = a1n.T.reshape(-1)
    out1, s1 = _make_s2(4)(srcp, dstp, a1f, _to_headmajor(h1), cm1)

    h2, a2n, cm2 = _t2(_from_headmajor(out1), s1.reshape(8, NP).T, h1, a1n,
                       cm1, b8(g1_b), g2_W.T, b8(g2_att_src[0]),
                       b8(g2_att_dst[0]))
    a2f = a2n.T.reshape(-1)
    out2, s2v = _make_s2(1)(srcp, dstp, a2f, _to_headmajor(h2), cm2)

    n2wTp = jnp.pad(n2_W.T, ((0, 0), (0, 126)))
    n2bp = jnp.pad(n2_b[None], ((0, 7), (0, 126)))
    wsT = e1_W[:, :64].T
    wdT = e1_W[:, 64:128].T
    npred_full, ps, pd = _t3(
        _from_headmajor(out2), s2v.reshape(8, NP).T, h2, a2n, cm2,
        b8(g2_b), n1_W.T, b8(n1_b), n2wTp, n2bp, wsT, wdT)

    e1pre = _make_s3()(srcp, dstp, ps, pd)
    weT = p8(e1_W[:, 128:].T)
    e2wTp = jnp.pad(e2_W.T, ((0, 0), (0, 127)))
    e2bp = jnp.pad(e2_b[None], ((0, 7), (0, 127)))
    epred_full = _t4(e1pre, attrp, weT, b8(e1_b), e2wTp, e2bp)

    edge_pred = epred_full.reshape(-1)[:E]
    node_pred = npred_full[:N, :2]
    return (edge_pred, node_pred)
